# baseline (speedup 1.0000x reference)
import jax
import jax.numpy as jnp
from jax import lax
from jax.experimental import pallas as pl
from jax.experimental.pallas import tpu as pltpu
from jax.experimental.pallas import tpu_sc as plsc

K = 4096
L = 16
NIMG = 16
NPIX = 512 * 512
HALF = NPIX // 2
CH = 16384
UNROLL = 32
NCH = HALF // CH
KP = K + 2 * L


def _body(pred_hbm, tgt_hbm, out_hbm, xa, xb, ta, tb, cnt_ref,
          cnt_sh, other_cnt, outv, sxa, sxb, sta, stb):
    c = lax.axis_index("c")
    s = lax.axis_index("s")
    img_local = s // 2
    half = s % 2
    img = c * 8 + img_local
    base = half * HALF

    zero = jnp.zeros((L,), jnp.float32)

    def zrow(i, _):
        cnt_ref[pl.ds(i * L, L)] = zero
        return 0
    lax.fori_loop(0, KP // L, zrow, 0)

    ones = jnp.ones((L,), jnp.float32)

    def start(bx, bt, ci, sx, st):
        off = base + ci * CH
        pltpu.async_copy(pred_hbm.at[img, pl.ds(off, CH)], bx, sx)
        pltpu.async_copy(tgt_hbm.at[img, pl.ds(off, CH)], bt, st)

    def wait(bx, bt, sx, st):
        pltpu.make_async_copy(pred_hbm.at[img, pl.ds(0, CH)], bx, sx).wait()
        pltpu.make_async_copy(tgt_hbm.at[img, pl.ds(0, CH)], bt, st).wait()

    def process(bx, bt, sv0):
        @plsc.parallel_loop(0, CH // L, 1, unroll=UNROLL, carry=sv0)
        def _inner(i, sv):
            x = bx[pl.ds(i * L, L)]
            t = bt[pl.ds(i * L, L)]
            p = 1.0 / (1.0 + jnp.exp(-x))
            b = jnp.minimum((p * K).astype(jnp.int32), K - 1)
            idx = jnp.where(t > 0, K, b)
            plsc.addupdate_scatter(cnt_ref, [idx], ones)
            return sv + jnp.where(t > 0, p, 0.0)
        return _inner

    start(xa, ta, 0, sxa, sta)

    def pair(pi, sv):
        c0 = pi * 2
        start(xb, tb, c0 + 1, sxb, stb)
        wait(xa, ta, sxa, sta)
        sv = process(xa, ta, sv)

        @pl.when(c0 + 2 < NCH)
        def _():
            start(xa, ta, c0 + 2, sxa, sta)
        wait(xb, tb, sxb, stb)
        sv = process(xb, tb, sv)
        return sv
    sv = lax.fori_loop(0, NCH // 2, pair, zero)
    cnt_ref[pl.ds(K + L, L)] = sv

    pltpu.sync_copy(cnt_ref, cnt_sh.at[img_local, half])
    plsc.subcore_barrier()

    @pl.when(half == 0)
    def _final():
        pltpu.sync_copy(cnt_sh.at[img_local, 1], other_cnt)
        n1 = (cnt_ref[pl.ds(K, L)] + other_cnt[pl.ds(K, L)])[0]
        s1 = jnp.sum(cnt_ref[pl.ds(K + L, L)] + other_cnt[pl.ds(K + L, L)])

        vzero = jnp.zeros((L,), jnp.float32)
        n1v = jnp.full((L,), n1)
        lanef = lax.iota(jnp.int32, L).astype(jnp.float32)
        invK = jnp.float32(1.0 / K)

        def scan(ci, carry):
            bcum, accv, topv = carry
            j0 = (K // L - 1 - ci) * L
            mc = cnt_ref[pl.ds(j0, L)] + other_cnt[pl.ds(j0, L)]
            rc = lax.rev(mc, (0,))
            midv = (jnp.float32(j0) + jnp.float32(L) - 0.5 - lanef) * invK
            rs = rc * midv
            bvec = bcum + lax.cumsum(rc, axis=0)
            avec = bvec - rc
            denom = jnp.maximum((n1v + avec) * (n1v + bvec), 1.0)
            accv = accv + n1v * rs / denom
            cond = (avec == 0.0) & (rc > 0.0)
            topv = jnp.where(cond, midv, topv)
            return (bcum + jnp.sum(rc), accv, topv)

        _, accv, topv = lax.fori_loop(
            0, K // L, scan, (jnp.float32(0.0), vzero, vzero))
        accs = jnp.where(n1 > 0.0, jnp.sum(accv), jnp.sum(topv))
        loss = 1.0 - s1 * jnp.float32(1.0 / NPIX) + accs
        outv[...] = jnp.full((L,), loss, jnp.float32)
        pltpu.sync_copy(outv, out_hbm.at[img])


def _build():
    mesh = plsc.VectorSubcoreMesh(core_axis_name="c", subcore_axis_name="s")
    return pl.kernel(
        _body,
        out_type=jax.ShapeDtypeStruct((NIMG, L), jnp.float32),
        mesh=mesh,
        compiler_params=pltpu.CompilerParams(needs_layout_passes=False),
        scratch_types=[
            pltpu.VMEM((CH,), jnp.float32),
            pltpu.VMEM((CH,), jnp.float32),
            pltpu.VMEM((CH,), jnp.int32),
            pltpu.VMEM((CH,), jnp.int32),
            pltpu.VMEM((KP,), jnp.float32),
            pltpu.VMEM_SHARED((10, 2, KP), jnp.float32),
            pltpu.VMEM((KP,), jnp.float32),
            pltpu.VMEM((L,), jnp.float32),
            pltpu.SemaphoreType.DMA,
            pltpu.SemaphoreType.DMA,
            pltpu.SemaphoreType.DMA,
            pltpu.SemaphoreType.DMA,
        ],
    )


def kernel(pred, target):
    predf = pred.reshape(NIMG, NPIX)
    tgtf = target.reshape(NIMG, NPIX)
    out = _build()(predf, tgtf)
    return jnp.mean(out[:, 0])


# speedup vs baseline: 1.1822x; 1.1822x over previous
import jax
import jax.numpy as jnp
from jax import lax
from jax.experimental import pallas as pl
from jax.experimental.pallas import tpu as pltpu
from jax.experimental.pallas import tpu_sc as plsc

K = 1024
L = 16
NIMG = 16
NPIX = 512 * 512
HALF = NPIX // 2
CH = 16384
UNROLL = 16
NCH = HALF // CH
KP = K + 2 * L


def _body(pred_hbm, tgt_hbm, out_hbm, xa, xb, ta, tb, cnt_ref,
          cnt_sh, other_cnt, outv, sxa, sxb, sta, stb):
    c = lax.axis_index("c")
    s = lax.axis_index("s")
    img_local = s // 2
    half = s % 2
    img = c * 8 + img_local
    base = half * HALF

    zero = jnp.zeros((L,), jnp.float32)

    def zrow(i, _):
        cnt_ref[pl.ds(i * L, L)] = zero
        return 0
    lax.fori_loop(0, KP // L, zrow, 0)

    ones = jnp.ones((L,), jnp.float32)

    def start(bx, bt, ci, sx, st):
        off = base + ci * CH
        pltpu.async_copy(pred_hbm.at[img, pl.ds(off, CH)], bx, sx)
        pltpu.async_copy(tgt_hbm.at[img, pl.ds(off, CH)], bt, st)

    def wait(bx, bt, sx, st):
        pltpu.make_async_copy(pred_hbm.at[img, pl.ds(0, CH)], bx, sx).wait()
        pltpu.make_async_copy(tgt_hbm.at[img, pl.ds(0, CH)], bt, st).wait()

    def process(bx, bt, sv0):
        @plsc.parallel_loop(0, CH // L, 1, unroll=UNROLL, carry=sv0)
        def _inner(i, sv):
            x = bx[pl.ds(i * L, L)]
            t = bt[pl.ds(i * L, L)]
            p = 1.0 / (1.0 + jnp.exp(-x))
            b = jnp.minimum((p * K).astype(jnp.int32), K - 1)
            idx = jnp.where(t > 0, K, b)
            plsc.addupdate_scatter(cnt_ref, [idx], ones)
            return sv + jnp.where(t > 0, p, 0.0)
        return _inner

    start(xa, ta, 0, sxa, sta)

    def pair(pi, sv):
        c0 = pi * 2
        start(xb, tb, c0 + 1, sxb, stb)
        wait(xa, ta, sxa, sta)
        sv = process(xa, ta, sv)

        @pl.when(c0 + 2 < NCH)
        def _():
            start(xa, ta, c0 + 2, sxa, sta)
        wait(xb, tb, sxb, stb)
        sv = process(xb, tb, sv)
        return sv
    sv = lax.fori_loop(0, NCH // 2, pair, zero)
    cnt_ref[pl.ds(K + L, L)] = sv

    pltpu.sync_copy(cnt_ref, cnt_sh.at[img_local, half])
    plsc.subcore_barrier()

    @pl.when(half == 0)
    def _final():
        pltpu.sync_copy(cnt_sh.at[img_local, 1], other_cnt)
        n1 = (cnt_ref[pl.ds(K, L)] + other_cnt[pl.ds(K, L)])[0]
        s1 = jnp.sum(cnt_ref[pl.ds(K + L, L)] + other_cnt[pl.ds(K + L, L)])

        vzero = jnp.zeros((L,), jnp.float32)
        n1v = jnp.full((L,), n1)
        lanef = lax.iota(jnp.int32, L).astype(jnp.float32)
        invK = jnp.float32(1.0 / K)

        def scan(ci, carry):
            bcum, accv, topv = carry
            j0 = (K // L - 1 - ci) * L
            mc = cnt_ref[pl.ds(j0, L)] + other_cnt[pl.ds(j0, L)]
            rc = lax.rev(mc, (0,))
            midv = (jnp.float32(j0) + jnp.float32(L) - 0.5 - lanef) * invK
            rs = rc * midv
            bvec = bcum + lax.cumsum(rc, axis=0)
            avec = bvec - rc
            denom = jnp.maximum((n1v + avec) * (n1v + bvec), 1.0)
            accv = accv + n1v * rs / denom
            cond = (avec == 0.0) & (rc > 0.0)
            topv = jnp.where(cond, midv, topv)
            return (bcum + jnp.sum(rc), accv, topv)

        _, accv, topv = lax.fori_loop(
            0, K // L, scan, (jnp.float32(0.0), vzero, vzero))
        accs = jnp.where(n1 > 0.0, jnp.sum(accv), jnp.sum(topv))
        loss = 1.0 - s1 * jnp.float32(1.0 / NPIX) + accs
        outv[...] = jnp.full((L,), loss, jnp.float32)
        pltpu.sync_copy(outv, out_hbm.at[img])


def _build():
    mesh = plsc.VectorSubcoreMesh(core_axis_name="c", subcore_axis_name="s")
    return pl.kernel(
        _body,
        out_type=jax.ShapeDtypeStruct((NIMG, L), jnp.float32),
        mesh=mesh,
        compiler_params=pltpu.CompilerParams(needs_layout_passes=False),
        scratch_types=[
            pltpu.VMEM((CH,), jnp.float32),
            pltpu.VMEM((CH,), jnp.float32),
            pltpu.VMEM((CH,), jnp.int32),
            pltpu.VMEM((CH,), jnp.int32),
            pltpu.VMEM((KP,), jnp.float32),
            pltpu.VMEM_SHARED((10, 2, KP), jnp.float32),
            pltpu.VMEM((KP,), jnp.float32),
            pltpu.VMEM((L,), jnp.float32),
            pltpu.SemaphoreType.DMA,
            pltpu.SemaphoreType.DMA,
            pltpu.SemaphoreType.DMA,
            pltpu.SemaphoreType.DMA,
        ],
    )


def kernel(pred, target):
    predf = pred.reshape(NIMG, NPIX)
    tgtf = target.reshape(NIMG, NPIX)
    out = _build()(predf, tgtf)
    return jnp.mean(out[:, 0])


# speedup vs baseline: 1.9678x; 1.6646x over previous
import jax
import jax.numpy as jnp
from jax import lax
from jax.experimental import pallas as pl
from jax.experimental.pallas import tpu as pltpu
from jax.experimental.pallas import tpu_sc as plsc

K = 4096
XLO = -13.0
SCALE = K / 26.0
L = 16
NIMG = 16
NPIX = 512 * 512
HALF = NPIX // 2
CH = 16384
UNROLL = 16
NCH = HALF // CH
KP = 2 * K + L


def _body(pred_hbm, tgt_hbm, out_hbm, xa, xb, ta, tb, cnt_ref,
          cnt_sh, other_cnt, outv, sxa, sxb, sta, stb):
    c = lax.axis_index("c")
    s = lax.axis_index("s")
    img_local = s // 2
    half = s % 2
    img = c * 8 + img_local
    base = half * HALF

    zero = jnp.zeros((L,), jnp.float32)

    def zrow(i, _):
        cnt_ref[pl.ds(i * L, L)] = zero
        return 0
    lax.fori_loop(0, KP // L, zrow, 0)

    ones = jnp.ones((L,), jnp.float32)

    def start(bx, bt, ci, sx, st):
        off = base + ci * CH
        pltpu.async_copy(pred_hbm.at[img, pl.ds(off, CH)], bx, sx)
        pltpu.async_copy(tgt_hbm.at[img, pl.ds(off, CH)], bt, st)

    def wait(bx, bt, sx, st):
        pltpu.make_async_copy(pred_hbm.at[img, pl.ds(0, CH)], bx, sx).wait()
        pltpu.make_async_copy(tgt_hbm.at[img, pl.ds(0, CH)], bt, st).wait()

    def process(bx, bt):
        @plsc.parallel_loop(0, CH // L, 1, unroll=UNROLL)
        def _inner(i):
            x = bx[pl.ds(i * L, L)]
            t = bt[pl.ds(i * L, L)]
            bi = ((x - XLO) * SCALE).astype(jnp.int32)
            bi = jnp.minimum(jnp.maximum(bi, 0), K - 1)
            idx = bi + jnp.where(t > 0, K, 0)
            plsc.addupdate_scatter(cnt_ref, [idx], ones)

    start(xa, ta, 0, sxa, sta)

    def pair(pi, _):
        c0 = pi * 2
        start(xb, tb, c0 + 1, sxb, stb)
        wait(xa, ta, sxa, sta)
        process(xa, ta)

        @pl.when(c0 + 2 < NCH)
        def _():
            start(xa, ta, c0 + 2, sxa, sta)
        wait(xb, tb, sxb, stb)
        process(xb, tb)
        return 0
    lax.fori_loop(0, NCH // 2, pair, 0)

    pltpu.sync_copy(cnt_ref, cnt_sh.at[img_local, half])
    plsc.subcore_barrier()

    @pl.when(half == 0)
    def _final():
        pltpu.sync_copy(cnt_sh.at[img_local, 1], other_cnt)

        vzero = jnp.zeros((L,), jnp.float32)
        lanef = lax.iota(jnp.int32, L).astype(jnp.float32)
        invS = jnp.float32(26.0 / K)

        def scan1(ci, carry):
            n1v, s1v = carry
            j0 = K + ci * L
            c1 = cnt_ref[pl.ds(j0, L)] + other_cnt[pl.ds(j0, L)]
            xc = XLO + (jnp.float32(ci * L) + lanef + 0.5) * invS
            pv = 1.0 / (1.0 + jnp.exp(-xc))
            return (n1v + c1, s1v + c1 * pv)

        n1v, s1v = lax.fori_loop(0, K // L, scan1, (vzero, vzero))
        n1 = jnp.sum(n1v)
        s1 = jnp.sum(s1v)
        n1v = jnp.full((L,), n1)

        def scan(ci, carry):
            bcum, accv, topv = carry
            j0 = (K // L - 1 - ci) * L
            mc = cnt_ref[pl.ds(j0, L)] + other_cnt[pl.ds(j0, L)]
            rc = lax.rev(mc, (0,))
            xc = XLO + (jnp.float32(j0) + jnp.float32(L) - 0.5 - lanef) * invS
            pv = 1.0 / (1.0 + jnp.exp(-xc))
            rs = rc * pv
            bvec = bcum + lax.cumsum(rc, axis=0)
            avec = bvec - rc
            denom = jnp.maximum((n1v + avec) * (n1v + bvec), 1.0)
            accv = accv + n1v * rs / denom
            cond = (avec == 0.0) & (rc > 0.0)
            topv = jnp.where(cond, pv, topv)
            return (bcum + jnp.sum(rc), accv, topv)

        _, accv, topv = lax.fori_loop(
            0, K // L, scan, (jnp.float32(0.0), vzero, vzero))
        accs = jnp.where(n1 > 0.0, jnp.sum(accv), jnp.sum(topv))
        loss = 1.0 - s1 * jnp.float32(1.0 / NPIX) + accs
        outv[...] = jnp.full((L,), loss, jnp.float32)
        pltpu.sync_copy(outv, out_hbm.at[img])


def _build():
    mesh = plsc.VectorSubcoreMesh(core_axis_name="c", subcore_axis_name="s")
    return pl.kernel(
        _body,
        out_type=jax.ShapeDtypeStruct((NIMG, L), jnp.float32),
        mesh=mesh,
        compiler_params=pltpu.CompilerParams(needs_layout_passes=False),
        scratch_types=[
            pltpu.VMEM((CH,), jnp.float32),
            pltpu.VMEM((CH,), jnp.float32),
            pltpu.VMEM((CH,), jnp.int32),
            pltpu.VMEM((CH,), jnp.int32),
            pltpu.VMEM((KP,), jnp.float32),
            pltpu.VMEM_SHARED((10, 2, KP), jnp.float32),
            pltpu.VMEM((KP,), jnp.float32),
            pltpu.VMEM((L,), jnp.float32),
            pltpu.SemaphoreType.DMA,
            pltpu.SemaphoreType.DMA,
            pltpu.SemaphoreType.DMA,
            pltpu.SemaphoreType.DMA,
        ],
    )


def kernel(pred, target):
    predf = pred.reshape(NIMG, NPIX)
    tgtf = target.reshape(NIMG, NPIX)
    out = _build()(predf, tgtf)
    return jnp.mean(out[:, 0])


# speedup vs baseline: 2.0913x; 1.0627x over previous
import jax
import jax.numpy as jnp
from jax import lax
from jax.experimental import pallas as pl
from jax.experimental.pallas import tpu as pltpu
from jax.experimental.pallas import tpu_sc as plsc

K = 4096
XLO = -13.0
SCALE = K / 26.0
L = 16
NIMG = 16
NPIX = 512 * 512
HALF = NPIX // 2
CH = 16384
UNROLL = 16
NCH = HALF // CH
KP = 2 * K + L


def _body(pred_hbm, tgt_hbm, out_hbm, xa, xb, ta, tb, cnt_ref,
          cnt_sh, other_cnt, outv, sxa, sxb, sta, stb):
    c = lax.axis_index("c")
    s = lax.axis_index("s")
    img_local = s // 2
    half = s % 2
    img = c * 8 + img_local
    base = half * HALF

    zero = jnp.zeros((L,), jnp.float32)

    def zrow(i, _):
        cnt_ref[pl.ds(i * L, L)] = zero
        return 0
    lax.fori_loop(0, KP // L, zrow, 0)

    ones = jnp.ones((L,), jnp.float32)

    def start(bx, bt, ci, sx, st):
        off = base + ci * CH
        pltpu.async_copy(pred_hbm.at[img, pl.ds(off, CH)], bx, sx)
        pltpu.async_copy(tgt_hbm.at[img, pl.ds(off, CH)], bt, st)

    def wait(bx, bt, sx, st):
        pltpu.make_async_copy(pred_hbm.at[img, pl.ds(0, CH)], bx, sx).wait()
        pltpu.make_async_copy(tgt_hbm.at[img, pl.ds(0, CH)], bt, st).wait()

    def process(bx, bt):
        @plsc.parallel_loop(0, CH // L, 1, unroll=UNROLL)
        def _inner(i):
            x = bx[pl.ds(i * L, L)]
            t = bt[pl.ds(i * L, L)]
            bi = ((x - XLO) * SCALE).astype(jnp.int32)
            bi = jnp.minimum(jnp.maximum(bi, 0), K - 1)
            idx = bi + jnp.where(t > 0, K, 0)
            plsc.store_scatter(cnt_ref, [idx], ones)

    start(xa, ta, 0, sxa, sta)

    def pair(pi, _):
        c0 = pi * 2
        start(xb, tb, c0 + 1, sxb, stb)
        wait(xa, ta, sxa, sta)
        process(xa, ta)

        @pl.when(c0 + 2 < NCH)
        def _():
            start(xa, ta, c0 + 2, sxa, sta)
        wait(xb, tb, sxb, stb)
        process(xb, tb)
        return 0
    lax.fori_loop(0, NCH // 2, pair, 0)

    pltpu.sync_copy(cnt_ref, cnt_sh.at[img_local, half])
    plsc.subcore_barrier()

    @pl.when(half == 0)
    def _final():
        pltpu.sync_copy(cnt_sh.at[img_local, 1], other_cnt)

        vzero = jnp.zeros((L,), jnp.float32)
        lanef = lax.iota(jnp.int32, L).astype(jnp.float32)
        invS = jnp.float32(26.0 / K)

        def scan1(ci, carry):
            n1v, s1v = carry
            j0 = K + ci * L
            c1 = cnt_ref[pl.ds(j0, L)] + other_cnt[pl.ds(j0, L)]
            xc = XLO + (jnp.float32(ci * L) + lanef + 0.5) * invS
            pv = 1.0 / (1.0 + jnp.exp(-xc))
            return (n1v + c1, s1v + c1 * pv)

        n1v, s1v = lax.fori_loop(0, K // L, scan1, (vzero, vzero))
        n1 = jnp.sum(n1v)
        s1 = jnp.sum(s1v)
        n1v = jnp.full((L,), n1)

        def scan(ci, carry):
            bcum, accv, topv = carry
            j0 = (K // L - 1 - ci) * L
            mc = cnt_ref[pl.ds(j0, L)] + other_cnt[pl.ds(j0, L)]
            rc = lax.rev(mc, (0,))
            xc = XLO + (jnp.float32(j0) + jnp.float32(L) - 0.5 - lanef) * invS
            pv = 1.0 / (1.0 + jnp.exp(-xc))
            rs = rc * pv
            bvec = bcum + lax.cumsum(rc, axis=0)
            avec = bvec - rc
            denom = jnp.maximum((n1v + avec) * (n1v + bvec), 1.0)
            accv = accv + n1v * rs / denom
            cond = (avec == 0.0) & (rc > 0.0)
            topv = jnp.where(cond, pv, topv)
            return (bcum + jnp.sum(rc), accv, topv)

        _, accv, topv = lax.fori_loop(
            0, K // L, scan, (jnp.float32(0.0), vzero, vzero))
        accs = jnp.where(n1 > 0.0, jnp.sum(accv), jnp.sum(topv))
        loss = 1.0 - s1 * jnp.float32(1.0 / NPIX) + accs
        outv[...] = jnp.full((L,), loss, jnp.float32)
        pltpu.sync_copy(outv, out_hbm.at[img])


def _build():
    mesh = plsc.VectorSubcoreMesh(core_axis_name="c", subcore_axis_name="s")
    return pl.kernel(
        _body,
        out_type=jax.ShapeDtypeStruct((NIMG, L), jnp.float32),
        mesh=mesh,
        compiler_params=pltpu.CompilerParams(needs_layout_passes=False),
        scratch_types=[
            pltpu.VMEM((CH,), jnp.float32),
            pltpu.VMEM((CH,), jnp.float32),
            pltpu.VMEM((CH,), jnp.int32),
            pltpu.VMEM((CH,), jnp.int32),
            pltpu.VMEM((KP,), jnp.float32),
            pltpu.VMEM_SHARED((10, 2, KP), jnp.float32),
            pltpu.VMEM((KP,), jnp.float32),
            pltpu.VMEM((L,), jnp.float32),
            pltpu.SemaphoreType.DMA,
            pltpu.SemaphoreType.DMA,
            pltpu.SemaphoreType.DMA,
            pltpu.SemaphoreType.DMA,
        ],
    )


def kernel(pred, target):
    predf = pred.reshape(NIMG, NPIX)
    tgtf = target.reshape(NIMG, NPIX)
    out = _build()(predf, tgtf)
    return jnp.mean(out[:, 0])
